# baseline (speedup 1.0000x reference)
import functools

import jax
import jax.numpy as jnp
from jax import lax
from jax.experimental import pallas as pl
from jax.experimental.pallas import tpu as pltpu
from jax.experimental.pallas import tpu_sc as plsc

N = 2048
D = 768
E = 8
K = 2
DFF = 3072
PROJ = 256

BB = 256
NK = N * K
MB = NK // BB + E
PB = MB * BB

NW = 32
CW = N // NW

CH = 128
NCH = N // CH


def _rt_body(x_ref, wp_ref, bp_ref, sim_ref, te_ref,
             dmat_ref, gate_ref, be_ref, used_ref):
    xb = x_ref[...]
    proj = lax.dot_general(xb, wp_ref[...], (((1,), (1,)), ((), ())),
                           preferred_element_type=jnp.float32)
    proj = proj + bp_ref[...]
    pn = jnp.sqrt(jnp.sum(proj * proj, axis=-1, keepdims=True))
    proj = proj / jnp.maximum(pn, 1e-12)
    sim = sim_ref[...]
    sn = jnp.sqrt(jnp.sum(sim * sim, axis=0, keepdims=True))
    sim = sim / jnp.maximum(sn, 1e-12)
    lg = lax.dot_general(proj, sim, (((1,), (0,)), ((), ())),
                         preferred_element_type=jnp.float32)
    lg = lg * jnp.exp(te_ref[0, 0])

    iot = lax.broadcasted_iota(jnp.int32, lg.shape, 1)
    m1 = jnp.max(lg, axis=-1, keepdims=True)
    i1 = jnp.min(jnp.where(lg == m1, iot, E), axis=-1, keepdims=True)
    lg2 = jnp.where(iot == i1, -jnp.inf, lg)
    m2 = jnp.max(lg2, axis=-1, keepdims=True)
    i2 = jnp.min(jnp.where(lg2 == m2, iot, E), axis=-1, keepdims=True)
    t = jnp.exp(m2 - m1)
    g1 = 1.0 / (1.0 + t)
    g2 = t / (1.0 + t)
    gate_ref[...] = jnp.concatenate([g1, g2], axis=1)

    oh0 = (iot == i1).astype(jnp.float32)
    oh1 = (iot == i2).astype(jnp.float32)
    s = oh0 + oh1

    ri = lax.broadcasted_iota(jnp.int32, (CH, CH), 0)
    ci = lax.broadcasted_iota(jnp.int32, (CH, CH), 1)
    tri = (ci < ri).astype(jnp.float32)
    chunks = []
    tots = []
    for b in range(NCH):
        chunk = s[b * CH:(b + 1) * CH, :]
        chunks.append(lax.dot_general(tri, chunk, (((1,), (0,)), ((), ())),
                                      preferred_element_type=jnp.float32))
        tots.append(jnp.sum(chunk, axis=0, keepdims=True))
    totals = jnp.concatenate(tots, axis=0)
    ri2 = lax.broadcasted_iota(jnp.int32, (NCH, NCH), 0)
    ci2 = lax.broadcasted_iota(jnp.int32, (NCH, NCH), 1)
    tri2 = (ci2 < ri2).astype(jnp.float32)
    offs = lax.dot_general(tri2, totals, (((1,), (0,)), ((), ())),
                           preferred_element_type=jnp.float32)
    cbr = jnp.concatenate(
        [chunks[b] + offs[b:b + 1, :] for b in range(NCH)], axis=0)

    counts = jnp.sum(totals, axis=0, keepdims=True)
    pc = jnp.floor((counts + (BB - 1)) * (1.0 / BB)) * BB
    rj = lax.broadcasted_iota(jnp.int32, (E, E), 0)
    cj = lax.broadcasted_iota(jnp.int32, (E, E), 1)
    u_strict = (rj < cj).astype(jnp.float32)
    poff = lax.dot_general(pc, u_strict, (((1,), (0,)), ((), ())),
                           preferred_element_type=jnp.float32)

    base = poff + cbr
    d0 = jnp.sum(oh0 * base, axis=1, keepdims=True)
    d1 = jnp.sum(oh1 * (base + oh0), axis=1, keepdims=True)
    dmat_ref[...] = jnp.concatenate(
        [d0.astype(jnp.int32), d1.astype(jnp.int32)], axis=1)

    nblk = pc * (1.0 / BB)
    u_incl = (rj <= cj).astype(jnp.float32)
    bcum = lax.dot_general(nblk, u_incl, (((1,), (0,)), ((), ())),
                           preferred_element_type=jnp.float32)
    used = bcum[0:1, E - 1:E].astype(jnp.int32)
    bcum_b = jnp.broadcast_to(bcum.astype(jnp.int32), (MB, E))
    barc = lax.broadcasted_iota(jnp.int32, (MB, E), 0)
    be_raw = jnp.sum((barc >= bcum_b).astype(jnp.int32), axis=1,
                     keepdims=True)
    bar1 = lax.broadcasted_iota(jnp.int32, (MB, 1), 0)
    bemin = jnp.minimum(be_raw, E - 1)
    in_use = bar1 < used[0, 0]
    last_e = jnp.max(jnp.where(in_use, bemin, -1))
    be_ref[...] = jnp.where(in_use, bemin, last_e)
    used_ref[...] = used


def _router_tables(x, Wp, bp, sim, temp):
    return pl.pallas_call(
        _rt_body,
        out_shape=[
            jax.ShapeDtypeStruct((N, K), jnp.int32),
            jax.ShapeDtypeStruct((N, K), jnp.float32),
            jax.ShapeDtypeStruct((MB, 1), jnp.int32),
            jax.ShapeDtypeStruct((1, 1), jnp.int32),
        ],
    )(x, Wp, bp.reshape(1, PROJ), sim, temp.reshape(1, 1))


def _sc_dispatch(x, pos0, pos1):
    mesh = plsc.VectorSubcoreMesh(core_axis_name="c", subcore_axis_name="s")

    @functools.partial(
        pl.kernel,
        mesh=mesh,
        out_type=jax.ShapeDtypeStruct((PB, D), jnp.float32),
        scratch_types=[
            pltpu.VMEM((CW,), jnp.int32),
            pltpu.VMEM((CW,), jnp.int32),
            pltpu.VMEM((CW, D), jnp.float32),
            pltpu.SemaphoreType.DMA,
            pltpu.SemaphoreType.DMA,
        ],
    )
    def k(x_hbm, p0_hbm, p1_hbm, out_hbm, ia_v, ib_v, x_v, sa, sb):
        wid = lax.axis_index("s") * 2 + lax.axis_index("c")
        base = wid * CW
        pltpu.sync_copy(p0_hbm.at[pl.ds(base, CW)], ia_v)
        pltpu.sync_copy(p1_hbm.at[pl.ds(base, CW)], ib_v)
        pltpu.sync_copy(x_hbm.at[pl.ds(base, CW)], x_v)
        ca = pltpu.async_copy(x_v, out_hbm.at[ia_v], sa)
        cb = pltpu.async_copy(x_v, out_hbm.at[ib_v], sb)
        ca.wait()
        cb.wait()

    return k(x, pos0, pos1)


def _ffn_body(be_ref, used_ref, xs_ref, w1_ref, b1_ref, w2_ref, b2_ref, g_ref,
              out_ref, w1c_ref, w2c_ref):
    b = pl.program_id(0)

    @pl.when(b < used_ref[0])
    def _():
        prev = be_ref[jnp.maximum(b - 1, 0)]
        fresh = jnp.logical_or(b == 0, be_ref[b] != prev)

        @pl.when(fresh)
        def _():
            w1c_ref[...] = w1_ref[0].astype(jnp.bfloat16)
            w2c_ref[...] = w2_ref[0].astype(jnp.bfloat16)

        xb = xs_ref[...].astype(jnp.bfloat16)
        h = lax.dot_general(xb, w1c_ref[...], (((1,), (1,)), ((), ())),
                            preferred_element_type=jnp.float32)
        h = jnp.maximum(h + b1_ref[0], 0.0).astype(jnp.bfloat16)
        y = lax.dot_general(h, w2c_ref[...], (((1,), (1,)), ((), ())),
                            preferred_element_type=jnp.float32)
        g = g_ref[...]
        out_ref[...] = jnp.where(g > 0.0, (y + b2_ref[0]) * g, 0.0)

    @pl.when(b >= used_ref[0])
    def _():
        out_ref[...] = jnp.zeros_like(out_ref)


def _ffn(xs, W1, b1, W2, b2, gate_sorted, be, used):
    grid_spec = pltpu.PrefetchScalarGridSpec(
        num_scalar_prefetch=2,
        grid=(MB,),
        in_specs=[
            pl.BlockSpec((BB, D), lambda b, be, used: (b, 0)),
            pl.BlockSpec((1, DFF, D), lambda b, be, used: (be[b], 0, 0)),
            pl.BlockSpec((1, 1, DFF), lambda b, be, used: (be[b], 0, 0)),
            pl.BlockSpec((1, D, DFF), lambda b, be, used: (be[b], 0, 0)),
            pl.BlockSpec((1, 1, D), lambda b, be, used: (be[b], 0, 0)),
            pl.BlockSpec((BB, 1), lambda b, be, used: (b, 0)),
        ],
        out_specs=pl.BlockSpec((BB, D), lambda b, be, used: (b, 0)),
        scratch_shapes=[
            pltpu.VMEM((DFF, D), jnp.bfloat16),
            pltpu.VMEM((D, DFF), jnp.bfloat16),
        ],
    )
    return pl.pallas_call(
        _ffn_body,
        grid_spec=grid_spec,
        out_shape=jax.ShapeDtypeStruct((PB, D), jnp.float32),
    )(be, used, xs, W1, b1.reshape(E, 1, DFF), W2, b2.reshape(E, 1, D),
      gate_sorted.reshape(PB, 1))


def _sc_combine(ys, pos0, pos1):
    mesh = plsc.VectorSubcoreMesh(core_axis_name="c", subcore_axis_name="s")

    @functools.partial(
        pl.kernel,
        mesh=mesh,
        out_type=[
            jax.ShapeDtypeStruct((N, D), jnp.float32),
            jax.ShapeDtypeStruct((N, D), jnp.float32),
        ],
        scratch_types=[
            pltpu.VMEM((CW,), jnp.int32),
            pltpu.VMEM((CW,), jnp.int32),
            pltpu.VMEM((CW, D), jnp.float32),
            pltpu.VMEM((CW, D), jnp.float32),
            pltpu.SemaphoreType.DMA,
            pltpu.SemaphoreType.DMA,
            pltpu.SemaphoreType.DMA,
            pltpu.SemaphoreType.DMA,
        ],
    )
    def k(ys_hbm, p0_hbm, p1_hbm, ya_hbm, yb_hbm, ia_v, ib_v, ra_v, rb_v,
          sa, sb, swa, swb):
        wid = lax.axis_index("s") * 2 + lax.axis_index("c")
        base = wid * CW
        pltpu.sync_copy(p0_hbm.at[pl.ds(base, CW)], ia_v)
        pltpu.sync_copy(p1_hbm.at[pl.ds(base, CW)], ib_v)
        ca = pltpu.async_copy(ys_hbm.at[ia_v], ra_v, sa)
        cb = pltpu.async_copy(ys_hbm.at[ib_v], rb_v, sb)
        ca.wait()
        wa = pltpu.async_copy(ra_v, ya_hbm.at[pl.ds(base, CW)], swa)
        cb.wait()
        wb = pltpu.async_copy(rb_v, yb_hbm.at[pl.ds(base, CW)], swb)
        wa.wait()
        wb.wait()

    return k(ys, pos0, pos1)


def _add_body(a_ref, b_ref, o_ref):
    o_ref[...] = a_ref[...] + b_ref[...]


def _add(ya, yb):
    BA = 512
    return pl.pallas_call(
        _add_body,
        grid=(N // BA,),
        in_specs=[
            pl.BlockSpec((BA, D), lambda b: (b, 0)),
            pl.BlockSpec((BA, D), lambda b: (b, 0)),
        ],
        out_specs=pl.BlockSpec((BA, D), lambda b: (b, 0)),
        out_shape=jax.ShapeDtypeStruct((N, D), jnp.float32),
    )(ya, yb)


def kernel(x, Wp, bp, sim, temp, W1, b1, W2, b2):
    dmat, gate, be2, used2 = _router_tables(x, Wp, bp, sim, temp)
    pos0 = dmat[:, 0]
    pos1 = dmat[:, 1]
    gate_sorted = jnp.zeros((PB,), jnp.float32).at[dmat.reshape(NK)].set(
        gate.reshape(NK))
    xs = _sc_dispatch(x, pos0, pos1)
    ys = _ffn(xs, W1, b1, W2, b2, gate_sorted, be2.reshape(MB), used2.reshape(1))
    ya, yb = _sc_combine(ys, pos0, pos1)
    return _add(ya, yb)


# speedup vs baseline: 1.0602x; 1.0602x over previous
import functools

import jax
import jax.numpy as jnp
from jax import lax
from jax.experimental import pallas as pl
from jax.experimental.pallas import tpu as pltpu
from jax.experimental.pallas import tpu_sc as plsc

N = 2048
D = 768
E = 8
K = 2
DFF = 3072
PROJ = 256

BB = 256
NK = N * K
MB = NK // BB + E
PB = MB * BB

NW = 32
CW = N // NW

CH = 128
NCH = N // CH


def _rt_body(x_ref, wp_ref, bp_ref, sim_ref, te_ref,
             dmat_ref, gate_ref, be_ref, used_ref, p0_ref, p1_ref):
    xb = x_ref[...]
    proj = lax.dot_general(xb, wp_ref[...], (((1,), (1,)), ((), ())),
                           preferred_element_type=jnp.float32)
    proj = proj + bp_ref[...]
    pn = jnp.sqrt(jnp.sum(proj * proj, axis=-1, keepdims=True))
    proj = proj / jnp.maximum(pn, 1e-12)
    sim = sim_ref[...]
    sn = jnp.sqrt(jnp.sum(sim * sim, axis=0, keepdims=True))
    sim = sim / jnp.maximum(sn, 1e-12)
    lg = lax.dot_general(proj, sim, (((1,), (0,)), ((), ())),
                         preferred_element_type=jnp.float32)
    lg = lg * jnp.exp(te_ref[0, 0])

    iot = lax.broadcasted_iota(jnp.int32, lg.shape, 1)
    m1 = jnp.max(lg, axis=-1, keepdims=True)
    i1 = jnp.min(jnp.where(lg == m1, iot, E), axis=-1, keepdims=True)
    lg2 = jnp.where(iot == i1, -jnp.inf, lg)
    m2 = jnp.max(lg2, axis=-1, keepdims=True)
    i2 = jnp.min(jnp.where(lg2 == m2, iot, E), axis=-1, keepdims=True)
    t = jnp.exp(m2 - m1)
    g1 = 1.0 / (1.0 + t)
    g2 = t / (1.0 + t)
    gate_ref[...] = jnp.concatenate([g1, g2], axis=1)

    oh0 = (iot == i1).astype(jnp.float32)
    oh1 = (iot == i2).astype(jnp.float32)
    s = oh0 + oh1

    ri = lax.broadcasted_iota(jnp.int32, (CH, CH), 0)
    ci = lax.broadcasted_iota(jnp.int32, (CH, CH), 1)
    tri = (ci < ri).astype(jnp.float32)
    chunks = []
    tots = []
    for b in range(NCH):
        chunk = s[b * CH:(b + 1) * CH, :]
        chunks.append(lax.dot_general(tri, chunk, (((1,), (0,)), ((), ())),
                                      preferred_element_type=jnp.float32))
        tots.append(jnp.sum(chunk, axis=0, keepdims=True))
    totals = jnp.concatenate(tots, axis=0)
    ri2 = lax.broadcasted_iota(jnp.int32, (NCH, NCH), 0)
    ci2 = lax.broadcasted_iota(jnp.int32, (NCH, NCH), 1)
    tri2 = (ci2 < ri2).astype(jnp.float32)
    offs = lax.dot_general(tri2, totals, (((1,), (0,)), ((), ())),
                           preferred_element_type=jnp.float32)
    cbr = jnp.concatenate(
        [chunks[b] + offs[b:b + 1, :] for b in range(NCH)], axis=0)

    counts = jnp.sum(totals, axis=0, keepdims=True)
    pc = jnp.floor((counts + (BB - 1)) * (1.0 / BB)) * BB
    rj = lax.broadcasted_iota(jnp.int32, (E, E), 0)
    cj = lax.broadcasted_iota(jnp.int32, (E, E), 1)
    u_strict = (rj < cj).astype(jnp.float32)
    poff = lax.dot_general(pc, u_strict, (((1,), (0,)), ((), ())),
                           preferred_element_type=jnp.float32)

    base = poff + cbr
    d0 = jnp.sum(oh0 * base, axis=1, keepdims=True)
    d1 = jnp.sum(oh1 * (base + oh0), axis=1, keepdims=True)
    dmat_ref[...] = jnp.concatenate(
        [d0.astype(jnp.int32), d1.astype(jnp.int32)], axis=1)
    p0_ref[...] = d0.astype(jnp.int32).reshape(N)
    p1_ref[...] = d1.astype(jnp.int32).reshape(N)

    nblk = pc * (1.0 / BB)
    u_incl = (rj <= cj).astype(jnp.float32)
    bcum = lax.dot_general(nblk, u_incl, (((1,), (0,)), ((), ())),
                           preferred_element_type=jnp.float32)
    used = bcum[0:1, E - 1:E].astype(jnp.int32)
    bcum_b = jnp.broadcast_to(bcum.astype(jnp.int32), (MB, E))
    barc = lax.broadcasted_iota(jnp.int32, (MB, E), 0)
    be_raw = jnp.sum((barc >= bcum_b).astype(jnp.int32), axis=1,
                     keepdims=True)
    bar1 = lax.broadcasted_iota(jnp.int32, (MB, 1), 0)
    bemin = jnp.minimum(be_raw, E - 1)
    in_use = bar1 < used[0, 0]
    last_e = jnp.max(jnp.where(in_use, bemin, -1))
    be_ref[...] = jnp.where(in_use, bemin, last_e)
    used_ref[...] = used


def _router_tables(x, Wp, bp, sim, temp):
    return pl.pallas_call(
        _rt_body,
        out_shape=[
            jax.ShapeDtypeStruct((N, K), jnp.int32),
            jax.ShapeDtypeStruct((N, K), jnp.float32),
            jax.ShapeDtypeStruct((MB, 1), jnp.int32),
            jax.ShapeDtypeStruct((1, 1), jnp.int32),
            jax.ShapeDtypeStruct((N,), jnp.int32),
            jax.ShapeDtypeStruct((N,), jnp.int32),
        ],
    )(x, Wp, bp.reshape(1, PROJ), sim, temp.reshape(1, 1))


def _sc_dispatch(x, pos0, pos1):
    mesh = plsc.VectorSubcoreMesh(core_axis_name="c", subcore_axis_name="s")

    @functools.partial(
        pl.kernel,
        mesh=mesh,
        out_type=jax.ShapeDtypeStruct((PB, D), jnp.float32),
        scratch_types=[
            pltpu.VMEM((CW,), jnp.int32),
            pltpu.VMEM((CW,), jnp.int32),
            pltpu.VMEM((CW, D), jnp.float32),
            pltpu.SemaphoreType.DMA,
            pltpu.SemaphoreType.DMA,
            pltpu.SemaphoreType.DMA,
            pltpu.SemaphoreType.DMA,
            pltpu.SemaphoreType.DMA,
        ],
    )
    def k(x_hbm, p0_hbm, p1_hbm, out_hbm, ia_v, ib_v, x_v, sa, sb, sx, sp0, sp1):
        wid = lax.axis_index("s") * 2 + lax.axis_index("c")
        base = wid * CW
        cx = pltpu.async_copy(x_hbm.at[pl.ds(base, CW)], x_v, sx)
        cp0 = pltpu.async_copy(p0_hbm.at[pl.ds(base, CW)], ia_v, sp0)
        cp1 = pltpu.async_copy(p1_hbm.at[pl.ds(base, CW)], ib_v, sp1)
        cx.wait()
        cp0.wait()
        cp1.wait()
        ca = pltpu.async_copy(x_v, out_hbm.at[ia_v], sa)
        cb = pltpu.async_copy(x_v, out_hbm.at[ib_v], sb)
        ca.wait()
        cb.wait()

    return k(x, pos0, pos1)


def _ffn_body(be_ref, used_ref, xs_ref, w1_ref, b1_ref, w2_ref, b2_ref, g_ref,
              out_ref):
    b = pl.program_id(0)

    @pl.when(b < used_ref[0])
    def _():
        xb = xs_ref[...]
        h = lax.dot_general(xb, w1_ref[0], (((1,), (1,)), ((), ())),
                            preferred_element_type=jnp.float32)
        h = jnp.maximum(h + b1_ref[0], 0.0)
        y = lax.dot_general(h, w2_ref[0], (((1,), (1,)), ((), ())),
                            preferred_element_type=jnp.float32)
        g = g_ref[...]
        out_ref[...] = jnp.where(g > 0.0, (y + b2_ref[0]) * g, 0.0)

    @pl.when(b >= used_ref[0])
    def _():
        out_ref[...] = jnp.zeros_like(out_ref)


def _ffn(xs, W1, b1, W2, b2, gate_sorted, be, used):
    grid_spec = pltpu.PrefetchScalarGridSpec(
        num_scalar_prefetch=2,
        grid=(MB,),
        in_specs=[
            pl.BlockSpec((BB, D), lambda b, be, used: (b, 0)),
            pl.BlockSpec((1, DFF, D), lambda b, be, used: (be[b], 0, 0)),
            pl.BlockSpec((1, 1, DFF), lambda b, be, used: (be[b], 0, 0)),
            pl.BlockSpec((1, D, DFF), lambda b, be, used: (be[b], 0, 0)),
            pl.BlockSpec((1, 1, D), lambda b, be, used: (be[b], 0, 0)),
            pl.BlockSpec((BB, 1), lambda b, be, used: (b, 0)),
        ],
        out_specs=pl.BlockSpec((BB, D), lambda b, be, used: (b, 0)),
    )
    return pl.pallas_call(
        _ffn_body,
        grid_spec=grid_spec,
        out_shape=jax.ShapeDtypeStruct((PB, D), jnp.float32),
    )(be, used, xs, W1, b1.reshape(E, 1, DFF), W2, b2.reshape(E, 1, D),
      gate_sorted.reshape(PB, 1))


def _sc_combine(ys, pos0, pos1):
    mesh = plsc.VectorSubcoreMesh(core_axis_name="c", subcore_axis_name="s")

    @functools.partial(
        pl.kernel,
        mesh=mesh,
        out_type=[
            jax.ShapeDtypeStruct((N, D), jnp.float32),
            jax.ShapeDtypeStruct((N, D), jnp.float32),
        ],
        scratch_types=[
            pltpu.VMEM((CW,), jnp.int32),
            pltpu.VMEM((CW,), jnp.int32),
            pltpu.VMEM((CW, D), jnp.float32),
            pltpu.VMEM((CW, D), jnp.float32),
            pltpu.SemaphoreType.DMA,
            pltpu.SemaphoreType.DMA,
            pltpu.SemaphoreType.DMA,
            pltpu.SemaphoreType.DMA,
        ],
    )
    def k(ys_hbm, p0_hbm, p1_hbm, ya_hbm, yb_hbm, ia_v, ib_v, ra_v, rb_v,
          sa, sb, swa, swb):
        wid = lax.axis_index("s") * 2 + lax.axis_index("c")
        base = wid * CW
        pltpu.sync_copy(p0_hbm.at[pl.ds(base, CW)], ia_v)
        pltpu.sync_copy(p1_hbm.at[pl.ds(base, CW)], ib_v)
        ca = pltpu.async_copy(ys_hbm.at[ia_v], ra_v, sa)
        cb = pltpu.async_copy(ys_hbm.at[ib_v], rb_v, sb)
        ca.wait()
        wa = pltpu.async_copy(ra_v, ya_hbm.at[pl.ds(base, CW)], swa)
        cb.wait()
        wb = pltpu.async_copy(rb_v, yb_hbm.at[pl.ds(base, CW)], swb)
        wa.wait()
        wb.wait()

    return k(ys, pos0, pos1)


def _add_body(a_ref, b_ref, o_ref):
    o_ref[...] = a_ref[...] + b_ref[...]


def _add(ya, yb):
    BA = 512
    return pl.pallas_call(
        _add_body,
        grid=(N // BA,),
        in_specs=[
            pl.BlockSpec((BA, D), lambda b: (b, 0)),
            pl.BlockSpec((BA, D), lambda b: (b, 0)),
        ],
        out_specs=pl.BlockSpec((BA, D), lambda b: (b, 0)),
        out_shape=jax.ShapeDtypeStruct((N, D), jnp.float32),
    )(ya, yb)


def kernel(x, Wp, bp, sim, temp, W1, b1, W2, b2):
    dmat, gate, be2, used2, pos0, pos1 = _router_tables(x, Wp, bp, sim, temp)
    gate_sorted = jnp.zeros((PB,), jnp.float32).at[dmat.reshape(NK)].set(
        gate.reshape(NK))
    xs = _sc_dispatch(x, pos0, pos1)
    ys = _ffn(xs, W1, b1, W2, b2, gate_sorted, be2.reshape(MB), used2.reshape(1))
    ya, yb = _sc_combine(ys, pos0, pos1)
    return _add(ya, yb)


# speedup vs baseline: 1.1521x; 1.0867x over previous
import functools

import jax
import jax.numpy as jnp
from jax import lax
from jax.experimental import pallas as pl
from jax.experimental.pallas import tpu as pltpu
from jax.experimental.pallas import tpu_sc as plsc

N = 2048
D = 768
E = 8
K = 2
DFF = 3072
PROJ = 256

BB = 256
NK = N * K
MB = NK // BB + E
PB = MB * BB

NW = 32
CW = N // NW

CH = 128
NCH = N // CH


def _rt_body(x_ref, wp_ref, bp_ref, sim_ref, te_ref,
             gate_ref, be_ref, used_ref, p0_ref, p1_ref):
    xb = x_ref[...]
    proj = lax.dot_general(xb, wp_ref[...], (((1,), (1,)), ((), ())),
                           preferred_element_type=jnp.float32)
    proj = proj + bp_ref[...]
    pn = jnp.sqrt(jnp.sum(proj * proj, axis=-1, keepdims=True))
    proj = proj / jnp.maximum(pn, 1e-12)
    sim = sim_ref[...]
    sn = jnp.sqrt(jnp.sum(sim * sim, axis=0, keepdims=True))
    sim = sim / jnp.maximum(sn, 1e-12)
    lg = lax.dot_general(proj, sim, (((1,), (0,)), ((), ())),
                         preferred_element_type=jnp.float32)
    lg = lg * jnp.exp(te_ref[0, 0])

    iot = lax.broadcasted_iota(jnp.int32, lg.shape, 1)
    m1 = jnp.max(lg, axis=-1, keepdims=True)
    i1 = jnp.min(jnp.where(lg == m1, iot, E), axis=-1, keepdims=True)
    lg2 = jnp.where(iot == i1, -jnp.inf, lg)
    m2 = jnp.max(lg2, axis=-1, keepdims=True)
    i2 = jnp.min(jnp.where(lg2 == m2, iot, E), axis=-1, keepdims=True)
    t = jnp.exp(m2 - m1)
    g1 = 1.0 / (1.0 + t)
    g2 = t / (1.0 + t)
    gate_ref[...] = jnp.concatenate([g1, g2], axis=1)

    oh0 = (iot == i1).astype(jnp.float32)
    oh1 = (iot == i2).astype(jnp.float32)
    s = oh0 + oh1

    ri = lax.broadcasted_iota(jnp.int32, (CH, CH), 0)
    ci = lax.broadcasted_iota(jnp.int32, (CH, CH), 1)
    tri = (ci < ri).astype(jnp.float32)
    chunks = []
    tots = []
    for b in range(NCH):
        chunk = s[b * CH:(b + 1) * CH, :]
        chunks.append(lax.dot_general(tri, chunk, (((1,), (0,)), ((), ())),
                                      preferred_element_type=jnp.float32))
        tots.append(jnp.sum(chunk, axis=0, keepdims=True))
    totals = jnp.concatenate(tots, axis=0)
    ri2 = lax.broadcasted_iota(jnp.int32, (NCH, NCH), 0)
    ci2 = lax.broadcasted_iota(jnp.int32, (NCH, NCH), 1)
    tri2 = (ci2 < ri2).astype(jnp.float32)
    offs = lax.dot_general(tri2, totals, (((1,), (0,)), ((), ())),
                           preferred_element_type=jnp.float32)
    cbr = jnp.concatenate(
        [chunks[b] + offs[b:b + 1, :] for b in range(NCH)], axis=0)

    counts = jnp.sum(totals, axis=0, keepdims=True)
    pc = jnp.floor((counts + (BB - 1)) * (1.0 / BB)) * BB
    rj = lax.broadcasted_iota(jnp.int32, (E, E), 0)
    cj = lax.broadcasted_iota(jnp.int32, (E, E), 1)
    u_strict = (rj < cj).astype(jnp.float32)
    poff = lax.dot_general(pc, u_strict, (((1,), (0,)), ((), ())),
                           preferred_element_type=jnp.float32)

    base = poff + cbr
    d0 = jnp.sum(oh0 * base, axis=1, keepdims=True)
    d1 = jnp.sum(oh1 * (base + oh0), axis=1, keepdims=True)
    p0_ref[...] = d0.astype(jnp.int32).reshape(N)
    p1_ref[...] = d1.astype(jnp.int32).reshape(N)

    nblk = pc * (1.0 / BB)
    u_incl = (rj <= cj).astype(jnp.float32)
    bcum = lax.dot_general(nblk, u_incl, (((1,), (0,)), ((), ())),
                           preferred_element_type=jnp.float32)
    used = bcum[0:1, E - 1:E].astype(jnp.int32)
    bcum_b = jnp.broadcast_to(bcum.astype(jnp.int32), (MB, E))
    barc = lax.broadcasted_iota(jnp.int32, (MB, E), 0)
    be_raw = jnp.sum((barc >= bcum_b).astype(jnp.int32), axis=1,
                     keepdims=True)
    bar1 = lax.broadcasted_iota(jnp.int32, (MB, 1), 0)
    bemin = jnp.minimum(be_raw, E - 1)
    in_use = bar1 < used[0, 0]
    last_e = jnp.max(jnp.where(in_use, bemin, -1))
    be_ref[...] = jnp.where(in_use, bemin, last_e)
    used_ref[...] = used


def _router_tables(x, Wp, bp, sim, temp):
    return pl.pallas_call(
        _rt_body,
        out_shape=[
            jax.ShapeDtypeStruct((N, K), jnp.float32),
            jax.ShapeDtypeStruct((MB, 1), jnp.int32),
            jax.ShapeDtypeStruct((1, 1), jnp.int32),
            jax.ShapeDtypeStruct((N,), jnp.int32),
            jax.ShapeDtypeStruct((N,), jnp.int32),
        ],
    )(x, Wp, bp.reshape(1, PROJ), sim, temp.reshape(1, 1))


def _sc_dispatch(x, pos0, pos1):
    mesh = plsc.VectorSubcoreMesh(core_axis_name="c", subcore_axis_name="s")

    @functools.partial(
        pl.kernel,
        mesh=mesh,
        out_type=jax.ShapeDtypeStruct((PB, D), jnp.float32),
        scratch_types=[
            pltpu.VMEM((CW,), jnp.int32),
            pltpu.VMEM((CW,), jnp.int32),
            pltpu.VMEM((CW, D), jnp.float32),
            pltpu.SemaphoreType.DMA,
            pltpu.SemaphoreType.DMA,
            pltpu.SemaphoreType.DMA,
            pltpu.SemaphoreType.DMA,
            pltpu.SemaphoreType.DMA,
        ],
    )
    def k(x_hbm, p0_hbm, p1_hbm, out_hbm, ia_v, ib_v, x_v, sa, sb, sx, sp0, sp1):
        wid = lax.axis_index("s") * 2 + lax.axis_index("c")
        base = wid * CW
        cx = pltpu.async_copy(x_hbm.at[pl.ds(base, CW)], x_v, sx)
        cp0 = pltpu.async_copy(p0_hbm.at[pl.ds(base, CW)], ia_v, sp0)
        cp1 = pltpu.async_copy(p1_hbm.at[pl.ds(base, CW)], ib_v, sp1)
        cx.wait()
        cp0.wait()
        cp1.wait()
        ca = pltpu.async_copy(x_v, out_hbm.at[ia_v], sa)
        cb = pltpu.async_copy(x_v, out_hbm.at[ib_v], sb)
        ca.wait()
        cb.wait()

    return k(x, pos0, pos1)


def _ffn_body(be_ref, used_ref, xs_ref, w1_ref, b1_ref, w2_ref, b2_ref,
              out_ref):
    b = pl.program_id(0)

    @pl.when(b < used_ref[0])
    def _():
        xb = xs_ref[...]
        h = lax.dot_general(xb, w1_ref[0], (((1,), (1,)), ((), ())),
                            preferred_element_type=jnp.float32)
        h = jnp.maximum(h + b1_ref[0], 0.0)
        y = lax.dot_general(h, w2_ref[0], (((1,), (1,)), ((), ())),
                            preferred_element_type=jnp.float32)
        out_ref[...] = y + b2_ref[0]


def _ffn(xs, W1, b1, W2, b2, be, used):
    grid_spec = pltpu.PrefetchScalarGridSpec(
        num_scalar_prefetch=2,
        grid=(MB,),
        in_specs=[
            pl.BlockSpec((BB, D), lambda b, be, used: (b, 0)),
            pl.BlockSpec((1, DFF, D), lambda b, be, used: (be[b], 0, 0)),
            pl.BlockSpec((1, 1, DFF), lambda b, be, used: (be[b], 0, 0)),
            pl.BlockSpec((1, D, DFF), lambda b, be, used: (be[b], 0, 0)),
            pl.BlockSpec((1, 1, D), lambda b, be, used: (be[b], 0, 0)),
        ],
        out_specs=pl.BlockSpec((BB, D), lambda b, be, used: (b, 0)),
    )
    return pl.pallas_call(
        _ffn_body,
        grid_spec=grid_spec,
        out_shape=jax.ShapeDtypeStruct((PB, D), jnp.float32),
    )(be, used, xs, W1, b1.reshape(E, 1, DFF), W2, b2.reshape(E, 1, D))


def _sc_combine(ys, pos0, pos1):
    mesh = plsc.VectorSubcoreMesh(core_axis_name="c", subcore_axis_name="s")

    @functools.partial(
        pl.kernel,
        mesh=mesh,
        out_type=[
            jax.ShapeDtypeStruct((N, D), jnp.float32),
            jax.ShapeDtypeStruct((N, D), jnp.float32),
        ],
        scratch_types=[
            pltpu.VMEM((CW,), jnp.int32),
            pltpu.VMEM((CW,), jnp.int32),
            pltpu.VMEM((CW, D), jnp.float32),
            pltpu.VMEM((CW, D), jnp.float32),
            pltpu.SemaphoreType.DMA,
            pltpu.SemaphoreType.DMA,
            pltpu.SemaphoreType.DMA,
            pltpu.SemaphoreType.DMA,
        ],
    )
    def k(ys_hbm, p0_hbm, p1_hbm, ya_hbm, yb_hbm, ia_v, ib_v, ra_v, rb_v,
          sa, sb, swa, swb):
        wid = lax.axis_index("s") * 2 + lax.axis_index("c")
        base = wid * CW
        pltpu.sync_copy(p0_hbm.at[pl.ds(base, CW)], ia_v)
        pltpu.sync_copy(p1_hbm.at[pl.ds(base, CW)], ib_v)
        ca = pltpu.async_copy(ys_hbm.at[ia_v], ra_v, sa)
        cb = pltpu.async_copy(ys_hbm.at[ib_v], rb_v, sb)
        ca.wait()
        wa = pltpu.async_copy(ra_v, ya_hbm.at[pl.ds(base, CW)], swa)
        cb.wait()
        wb = pltpu.async_copy(rb_v, yb_hbm.at[pl.ds(base, CW)], swb)
        wa.wait()
        wb.wait()

    return k(ys, pos0, pos1)


def _add_body(a_ref, b_ref, g_ref, o_ref):
    g = g_ref[...]
    o_ref[...] = a_ref[...] * g[:, 0:1] + b_ref[...] * g[:, 1:2]


def _add(ya, yb, gate):
    BA = 512
    return pl.pallas_call(
        _add_body,
        grid=(N // BA,),
        in_specs=[
            pl.BlockSpec((BA, D), lambda b: (b, 0)),
            pl.BlockSpec((BA, D), lambda b: (b, 0)),
            pl.BlockSpec((BA, K), lambda b: (b, 0)),
        ],
        out_specs=pl.BlockSpec((BA, D), lambda b: (b, 0)),
        out_shape=jax.ShapeDtypeStruct((N, D), jnp.float32),
    )(ya, yb, gate)


def kernel(x, Wp, bp, sim, temp, W1, b1, W2, b2):
    gate, be2, used2, pos0, pos1 = _router_tables(x, Wp, bp, sim, temp)
    xs = _sc_dispatch(x, pos0, pos1)
    ys = _ffn(xs, W1, b1, W2, b2, be2.reshape(MB), used2.reshape(1))
    ya, yb = _sc_combine(ys, pos0, pos1)
    return _add(ya, yb, gate)
